# baseline (speedup 1.0000x reference)
import jax
import jax.numpy as jnp
from jax import lax
from jax.experimental import pallas as pl
from jax.experimental.pallas import tpu as pltpu
from jax.experimental.pallas import tpu_sc as plsc

_B = 16384
_D = 64
_L = 16
_NC = 2
_NS = 16
_NW = _NC * _NS
_BPW = _B // _NW
_CB = 64
_NCH = _BPW // _CB
_GAMMA = 1.0


def _tran_e_body(sidx, entP, relP, out,
                 vh, vt, vr, wh, wt, wr,
                 a1, a2, a3, a4, a5, a6,
                 b1, b2, b3, b4, b5, b6,
                 accv, semA, semB):
    wid = lax.axis_index("s") * _NC + lax.axis_index("c")
    pltpu.sync_copy(sidx.at[0, wid], vh)
    pltpu.sync_copy(sidx.at[1, wid], vt)
    pltpu.sync_copy(sidx.at[2, wid], vr)
    pltpu.sync_copy(sidx.at[3, wid], wh)
    pltpu.sync_copy(sidx.at[4, wid], wt)
    pltpu.sync_copy(sidx.at[5, wid], wr)

    def fire(c, c1, c2, c3, c4, c5, c6, sem):
        pltpu.async_copy(entP.at[vh.at[c]], c1, sem)
        pltpu.async_copy(entP.at[vt.at[c]], c2, sem)
        pltpu.async_copy(relP.at[vr.at[c]], c3, sem)
        pltpu.async_copy(entP.at[wh.at[c]], c4, sem)
        pltpu.async_copy(entP.at[wt.at[c]], c5, sem)
        pltpu.async_copy(relP.at[wr.at[c]], c6, sem)

    def drain(c1, c2, c3, c4, c5, c6, sem):
        src = entP.at[pl.ds(0, _CB)]
        pltpu.make_async_copy(src, c1, sem).wait()
        pltpu.make_async_copy(src, c2, sem).wait()
        pltpu.make_async_copy(src, c3, sem).wait()
        pltpu.make_async_copy(src, c4, sem).wait()
        pltpu.make_async_copy(src, c5, sem).wait()
        pltpu.make_async_copy(src, c6, sem).wait()

    def contrib(c, c1, c2, c3, c4, c5, c6, wsum0):
        def subgroup(sg, wsum):
            row0 = sg * _L
            for k in range(_L):
                row = row0 + k
                pv = jnp.zeros((_L,), jnp.float32)
                nv = jnp.zeros((_L,), jnp.float32)
                for m in range(_D // _L):
                    o = m * _L
                    sl = pl.ds(o, _L)
                    pv = pv + jnp.abs(c1[row, sl] + c3[row, sl] - c2[row, sl])
                    nv = nv + jnp.abs(c4[row, sl] + c6[row, sl] - c5[row, sl])
                wsum = wsum + jnp.maximum(
                    _GAMMA + jnp.sum(pv) - jnp.sum(nv), 0.0)
            return wsum

        return lax.fori_loop(0, _CB // _L, subgroup, wsum0)

    fire(0, a1, a2, a3, a4, a5, a6, semA)

    def chunk_pair(i, wsum):
        ca = 2 * i
        fire(ca + 1, b1, b2, b3, b4, b5, b6, semB)
        drain(a1, a2, a3, a4, a5, a6, semA)
        wsum = contrib(ca, a1, a2, a3, a4, a5, a6, wsum)
        fire(ca + 2, a1, a2, a3, a4, a5, a6, semA)
        drain(b1, b2, b3, b4, b5, b6, semB)
        return contrib(ca + 1, b1, b2, b3, b4, b5, b6, wsum)

    wsum = lax.fori_loop(0, _NCH // 2 - 1, chunk_pair, jnp.float32(0.0))

    fire(_NCH - 1, b1, b2, b3, b4, b5, b6, semB)
    drain(a1, a2, a3, a4, a5, a6, semA)
    wsum = contrib(_NCH - 2, a1, a2, a3, a4, a5, a6, wsum)
    drain(b1, b2, b3, b4, b5, b6, semB)
    wsum = contrib(_NCH - 1, b1, b2, b3, b4, b5, b6, wsum)

    lane = lax.iota(jnp.int32, _L)
    accv[...] = jnp.where(lane == 0, wsum, 0.0)
    pltpu.sync_copy(accv, out.at[wid])


_idx32 = [pltpu.VMEM((_NCH, _CB), jnp.int32)] * 6
_rowbuf = [pltpu.VMEM((_CB, 2 * _D), jnp.float32)] * 12

_sc_tran_e = pl.kernel(
    _tran_e_body,
    out_type=jax.ShapeDtypeStruct((_NW, _L), jnp.float32),
    mesh=plsc.VectorSubcoreMesh(core_axis_name="c", subcore_axis_name="s"),
    compiler_params=pltpu.CompilerParams(needs_layout_passes=False),
    scratch_types=[*_idx32, *_rowbuf,
                   pltpu.VMEM((_L,), jnp.float32),
                   pltpu.SemaphoreType.DMA,
                   pltpu.SemaphoreType.DMA],
)


_E = 1000000
_CTC = 32768
_TCG = (_E + _CTC - 1) // _CTC


def _pack_body(src_ref, dst_ref):
    dst_ref[:, pl.ds(0, _D)] = src_ref[...].T


_tc_pack = pl.pallas_call(
    _pack_body,
    grid=(_TCG,),
    in_specs=[pl.BlockSpec((_D, _CTC), lambda i: (0, i))],
    out_specs=pl.BlockSpec((_CTC, 2 * _D), lambda i: (i, 0)),
    out_shape=jax.ShapeDtypeStruct((_E, 2 * _D), jnp.float32),
)


def kernel(pos_head, pos_tail, pos_relation, neg_head, neg_tail, neg_relation,
           entity_embedding, relation_embedding):
    srcs = (pos_head, pos_tail, pos_relation,
            neg_head, neg_tail, neg_relation)
    sidx = jnp.stack(srcs).astype(jnp.int32).reshape(6, _NW, _NCH, _CB)
    entP = _tc_pack(entity_embedding.T)
    relP = jnp.pad(relation_embedding, ((0, 0), (0, _D)))
    partials = _sc_tran_e(sidx, entP, relP)
    return jnp.sum(partials)
